# Initial kernel scaffold; baseline (speedup 1.0000x reference)
import jax
import jax.numpy as jnp
from jax.experimental import pallas as pl


def kernel(x, edge_index, W1, b1, W2, b2):
    raise NotImplementedError("write your pallas kernel here")


# speedup vs baseline: 13.0421x; 13.0421x over previous
import functools

import jax
import jax.numpy as jnp
from jax import lax
from jax.experimental import pallas as pl
from jax.experimental.pallas import tpu as pltpu
from jax.experimental.pallas import tpu_sc as plsc

N = 10000
E = 320000
NTILES = 32
CHUNK = 80
EPT = E // NTILES
NCH = EPT // CHUNK
NPAD = 10240
RPT = NPAD // 16
BLK = 1000


def _make_agg(feat):
    mesh = plsc.VectorSubcoreMesh(core_axis_name="c", subcore_axis_name="s")

    @functools.partial(
        pl.kernel,
        out_type=jax.ShapeDtypeStruct((2, NPAD, feat), jnp.float32),
        mesh=mesh,
        scratch_types=[
            pltpu.VMEM((CHUNK,), jnp.int32),
            pltpu.VMEM((CHUNK,), jnp.int32),
            pltpu.VMEM((CHUNK, feat), jnp.float32),
            pltpu.VMEM_SHARED((NPAD, feat), jnp.float32),
        ],
        compiler_params=pltpu.CompilerParams(use_tc_tiling_on_sc=False),
    )
    def agg(y_hbm, row_hbm, col_hbm, z_hbm, out_hbm, gidx, sidx, rows, acc):
        cid = lax.axis_index("c")
        sid = lax.axis_index("s")
        r0 = sid * RPT
        pltpu.sync_copy(z_hbm.at[pl.ds(r0, RPT)], acc.at[pl.ds(r0, RPT)])
        plsc.subcore_barrier()
        base = (cid * 16 + sid) * EPT

        @pl.loop(0, NCH)
        def _(j):
            off = base + j * CHUNK
            pltpu.sync_copy(row_hbm.at[pl.ds(off, CHUNK)], gidx)
            pltpu.sync_copy(col_hbm.at[pl.ds(off, CHUNK)], sidx)
            pltpu.sync_copy(y_hbm.at[gidx], rows)
            pltpu.sync_copy(rows, acc.at[sidx], add=True)

        plsc.subcore_barrier()
        pltpu.sync_copy(acc.at[pl.ds(r0, RPT)], out_hbm.at[cid, pl.ds(r0, RPT)])

    return agg


def _f32_dot(a, b):
    return jnp.dot(a, b, preferred_element_type=jnp.float32)


def _dinv(d_ref):
    d = (d_ref[0] + d_ref[1] + 1.0)[:, 0:1]
    return lax.rsqrt(d)


def _tc_xw1(x, W1, degp):
    def body(x_ref, w_ref, d_ref, o_ref):
        o_ref[...] = _f32_dot(x_ref[...], w_ref[...]) * _dinv(d_ref)

    return pl.pallas_call(
        body,
        grid=(N // BLK,),
        in_specs=[
            pl.BlockSpec((BLK, 128), lambda i: (i, 0)),
            pl.BlockSpec((128, 64), lambda i: (0, 0)),
            pl.BlockSpec((2, BLK, 16), lambda i: (0, i, 0)),
        ],
        out_specs=pl.BlockSpec((BLK, 64), lambda i: (i, 0)),
        out_shape=jax.ShapeDtypeStruct((N, 64), jnp.float32),
    )(x, W1, degp)


def _tc_mid(aggp, y1, degp, b1r, W2p):
    def body(a_ref, y_ref, d_ref, b_ref, w_ref, o_ref):
        dinv = _dinv(d_ref)
        h = jnp.maximum(dinv * (a_ref[0] + a_ref[1] + y_ref[...]) + b_ref[...],
                        0.0)
        o_ref[...] = dinv * _f32_dot(h, w_ref[...])

    return pl.pallas_call(
        body,
        grid=(N // BLK,),
        in_specs=[
            pl.BlockSpec((2, BLK, 64), lambda i: (0, i, 0)),
            pl.BlockSpec((BLK, 64), lambda i: (i, 0)),
            pl.BlockSpec((2, BLK, 16), lambda i: (0, i, 0)),
            pl.BlockSpec((1, 64), lambda i: (0, 0)),
            pl.BlockSpec((64, 16), lambda i: (0, 0)),
        ],
        out_specs=pl.BlockSpec((BLK, 16), lambda i: (i, 0)),
        out_shape=jax.ShapeDtypeStruct((N, 16), jnp.float32),
    )(aggp, y1, degp, b1r, W2p)


def _tc_fin(agg2p, y2p, degp, b2p):
    def body(a_ref, y_ref, d_ref, b_ref, o_ref):
        s = _dinv(d_ref) * (a_ref[0] + a_ref[1] + y_ref[...]) + b_ref[...]
        o_ref[...] = s[:, 0:2]

    return pl.pallas_call(
        body,
        grid=(N // BLK,),
        in_specs=[
            pl.BlockSpec((2, BLK, 16), lambda i: (0, i, 0)),
            pl.BlockSpec((BLK, 16), lambda i: (i, 0)),
            pl.BlockSpec((2, BLK, 16), lambda i: (0, i, 0)),
            pl.BlockSpec((1, 16), lambda i: (0, 0)),
        ],
        out_specs=pl.BlockSpec((BLK, 2), lambda i: (i, 0)),
        out_shape=jax.ShapeDtypeStruct((N, 2), jnp.float32),
    )(agg2p, y2p, degp, b2p)


def kernel(x, edge_index, W1, b1, W2, b2):
    row = edge_index[0].astype(jnp.int32)
    col = edge_index[1].astype(jnp.int32)

    ones16 = jnp.ones((N, 16), jnp.float32)
    z16 = jnp.zeros((NPAD, 16), jnp.float32)
    z64 = jnp.zeros((NPAD, 64), jnp.float32)
    W2p = jnp.zeros((64, 16), jnp.float32).at[:, 0:2].set(W2)
    b1r = b1.reshape(1, 64)
    b2p = jnp.zeros((1, 16), jnp.float32).at[0, 0:2].set(b2)

    degp = _make_agg(16)(ones16, col, col, z16)
    y1 = _tc_xw1(x, W1, degp)
    agg1 = _make_agg(64)(y1, row, col, z64)
    y2p = _tc_mid(agg1, y1, degp, b1r, W2p)
    agg2 = _make_agg(16)(y2p, row, col, z16)
    return _tc_fin(agg2, y2p, degp, b2p)


# speedup vs baseline: 37.9048x; 2.9063x over previous
import functools

import jax
import jax.numpy as jnp
from jax import lax
from jax.experimental import pallas as pl
from jax.experimental.pallas import tpu as pltpu
from jax.experimental.pallas import tpu_sc as plsc

N = 10000
E = 320000
NTILES = 32
CHUNK = 80
EPT = E // NTILES
NCH = EPT // CHUNK
NPAD = 10240
RPT = NPAD // 16
BLK = 1000


def _make_agg(feat):
    mesh = plsc.VectorSubcoreMesh(core_axis_name="c", subcore_axis_name="s")

    @functools.partial(
        pl.kernel,
        out_type=jax.ShapeDtypeStruct((2, NPAD, feat), jnp.float32),
        mesh=mesh,
        scratch_types=[
            pltpu.VMEM((NCH, CHUNK), jnp.int32),
            pltpu.VMEM((NCH, CHUNK), jnp.int32),
            pltpu.VMEM((CHUNK, feat), jnp.float32),
            pltpu.VMEM((CHUNK, feat), jnp.float32),
            pltpu.VMEM_SHARED((NPAD, feat), jnp.float32),
            pltpu.SemaphoreType.DMA,
            pltpu.SemaphoreType.DMA,
        ],
        compiler_params=pltpu.CompilerParams(use_tc_tiling_on_sc=False),
    )
    def agg(y_hbm, row_hbm, col_hbm, z_hbm, out_hbm,
            ridx, sidx, buf0, buf1, acc, sem0, sem1):
        cid = lax.axis_index("c")
        sid = lax.axis_index("s")
        r0 = sid * RPT
        pltpu.sync_copy(z_hbm.at[pl.ds(r0, RPT)], acc.at[pl.ds(r0, RPT)])
        tile = cid * 16 + sid
        pltpu.sync_copy(row_hbm.at[pl.ds(tile * NCH, NCH)], ridx)
        pltpu.sync_copy(col_hbm.at[pl.ds(tile * NCH, NCH)], sidx)
        plsc.subcore_barrier()

        bufs = (buf0, buf1)
        sems = (sem0, sem1)

        def start(j, b):
            pltpu.async_copy(y_hbm.at[ridx.at[j]], bufs[b], sems[b])

        def finish(j, b):
            pltpu.make_async_copy(y_hbm.at[ridx.at[j]], bufs[b], sems[b]).wait()
            pltpu.sync_copy(bufs[b], acc.at[sidx.at[j]], add=True)

        start(0, 0)

        @pl.loop(0, (NCH - 1) // 2)
        def _(k):
            j = 2 * k
            start(j + 1, 1)
            finish(j, 0)
            start(j + 2, 0)
            finish(j + 1, 1)

        finish(NCH - 1, 0)

        plsc.subcore_barrier()
        pltpu.sync_copy(acc.at[pl.ds(r0, RPT)], out_hbm.at[cid, pl.ds(r0, RPT)])

    return agg


def _deg_hist(col2, z16):
    mesh = plsc.VectorSubcoreMesh(core_axis_name="c", subcore_axis_name="s")

    @functools.partial(
        pl.kernel,
        out_type=jax.ShapeDtypeStruct((2, NPAD, 16), jnp.float32),
        mesh=mesh,
        scratch_types=[
            pltpu.VMEM((NCH, CHUNK), jnp.int32),
            pltpu.VMEM((CHUNK, 16), jnp.float32),
            pltpu.VMEM_SHARED((NPAD, 16), jnp.float32),
        ],
        compiler_params=pltpu.CompilerParams(use_tc_tiling_on_sc=False),
    )
    def deg(col_hbm, z_hbm, out_hbm, sidx, ones, acc):
        cid = lax.axis_index("c")
        sid = lax.axis_index("s")
        r0 = sid * RPT
        pltpu.sync_copy(z_hbm.at[pl.ds(r0, RPT)], acc.at[pl.ds(r0, RPT)])
        tile = cid * 16 + sid
        pltpu.sync_copy(col_hbm.at[pl.ds(tile * NCH, NCH)], sidx)

        @pl.loop(0, CHUNK)
        def _(i):
            ones[i, :] = jnp.ones((16,), jnp.float32)

        plsc.subcore_barrier()

        @pl.loop(0, NCH)
        def _(j):
            pltpu.sync_copy(ones, acc.at[sidx.at[j]], add=True)

        plsc.subcore_barrier()
        pltpu.sync_copy(acc.at[pl.ds(r0, RPT)], out_hbm.at[cid, pl.ds(r0, RPT)])

    return deg(col2, z16)


def _f32_dot(a, b):
    return jnp.dot(a, b, preferred_element_type=jnp.float32)


def _dinv(d_ref):
    d = (d_ref[0] + d_ref[1] + 1.0)[:, 0:1]
    return lax.rsqrt(d)


def _tc_xw1(x, W1, degp):
    def body(x_ref, w_ref, d_ref, o_ref):
        o_ref[...] = _f32_dot(x_ref[...], w_ref[...]) * _dinv(d_ref)

    return pl.pallas_call(
        body,
        grid=(N // BLK,),
        in_specs=[
            pl.BlockSpec((BLK, 128), lambda i: (i, 0)),
            pl.BlockSpec((128, 64), lambda i: (0, 0)),
            pl.BlockSpec((2, BLK, 16), lambda i: (0, i, 0)),
        ],
        out_specs=pl.BlockSpec((BLK, 64), lambda i: (i, 0)),
        out_shape=jax.ShapeDtypeStruct((N, 64), jnp.float32),
    )(x, W1, degp)


def _tc_mid(aggp, y1, degp, b1r, W2p):
    def body(a_ref, y_ref, d_ref, b_ref, w_ref, o_ref):
        dinv = _dinv(d_ref)
        h = jnp.maximum(dinv * (a_ref[0] + a_ref[1] + y_ref[...]) + b_ref[...],
                        0.0)
        o_ref[...] = dinv * _f32_dot(h, w_ref[...])

    return pl.pallas_call(
        body,
        grid=(N // BLK,),
        in_specs=[
            pl.BlockSpec((2, BLK, 64), lambda i: (0, i, 0)),
            pl.BlockSpec((BLK, 64), lambda i: (i, 0)),
            pl.BlockSpec((2, BLK, 16), lambda i: (0, i, 0)),
            pl.BlockSpec((1, 64), lambda i: (0, 0)),
            pl.BlockSpec((64, 16), lambda i: (0, 0)),
        ],
        out_specs=pl.BlockSpec((BLK, 16), lambda i: (i, 0)),
        out_shape=jax.ShapeDtypeStruct((N, 16), jnp.float32),
    )(aggp, y1, degp, b1r, W2p)


def _tc_fin(agg2p, y2p, degp, b2p):
    def body(a_ref, y_ref, d_ref, b_ref, o_ref):
        s = _dinv(d_ref) * (a_ref[0] + a_ref[1] + y_ref[...]) + b_ref[...]
        o_ref[...] = s[:, 0:2]

    return pl.pallas_call(
        body,
        grid=(N // BLK,),
        in_specs=[
            pl.BlockSpec((2, BLK, 16), lambda i: (0, i, 0)),
            pl.BlockSpec((BLK, 16), lambda i: (i, 0)),
            pl.BlockSpec((2, BLK, 16), lambda i: (0, i, 0)),
            pl.BlockSpec((1, 16), lambda i: (0, 0)),
        ],
        out_specs=pl.BlockSpec((BLK, 2), lambda i: (i, 0)),
        out_shape=jax.ShapeDtypeStruct((N, 2), jnp.float32),
    )(agg2p, y2p, degp, b2p)


def kernel(x, edge_index, W1, b1, W2, b2):
    row2 = edge_index[0].astype(jnp.int32).reshape(E // CHUNK, CHUNK)
    col2 = edge_index[1].astype(jnp.int32).reshape(E // CHUNK, CHUNK)

    z16 = jnp.zeros((NPAD, 16), jnp.float32)
    z64 = jnp.zeros((NPAD, 64), jnp.float32)
    W2p = jnp.zeros((64, 16), jnp.float32).at[:, 0:2].set(W2)
    b1r = b1.reshape(1, 64)
    b2p = jnp.zeros((1, 16), jnp.float32).at[0, 0:2].set(b2)

    degp = _deg_hist(col2, z16)
    y1 = _tc_xw1(x, W1, degp)
    agg1 = _make_agg(64)(y1, row2, col2, z64)
    y2p = _tc_mid(agg1, y1, degp, b1r, W2p)
    agg2 = _make_agg(16)(y2p, row2, col2, z16)
    return _tc_fin(agg2, y2p, degp, b2p)


# speedup vs baseline: 47.3093x; 1.2481x over previous
import functools

import jax
import jax.numpy as jnp
from jax import lax
from jax.experimental import pallas as pl
from jax.experimental.pallas import tpu as pltpu
from jax.experimental.pallas import tpu_sc as plsc

N = 10000
E = 320000
NTILES = 32
CHUNK = 80
EPT = E // NTILES
NCH = EPT // CHUNK
NBUF = 5
NGRP = NCH // NBUF
NPAD = 10240
RPT = NPAD // 16
BLK = 1000


def _make_agg(feat):
    mesh = plsc.VectorSubcoreMesh(core_axis_name="c", subcore_axis_name="s")

    @functools.partial(
        pl.kernel,
        out_type=jax.ShapeDtypeStruct((2, NPAD, feat), jnp.float32),
        mesh=mesh,
        scratch_types=[
            pltpu.VMEM((NCH, CHUNK), jnp.int32),
            pltpu.VMEM((NCH, CHUNK), jnp.int32),
        ] + [pltpu.VMEM((CHUNK, feat), jnp.float32)] * NBUF
          + [pltpu.SemaphoreType.DMA] * NBUF
          + [pltpu.SemaphoreType.DMA] * NBUF
          + [pltpu.VMEM_SHARED((NPAD, feat), jnp.float32)],
        compiler_params=pltpu.CompilerParams(use_tc_tiling_on_sc=False),
    )
    def agg(y_hbm, row_hbm, col_hbm, z_hbm, out_hbm, ridx, sidx, *scr):
        bufs = scr[:NBUF]
        gsem = scr[NBUF:2 * NBUF]
        ssem = scr[2 * NBUF:3 * NBUF]
        acc = scr[3 * NBUF]
        cid = lax.axis_index("c")
        sid = lax.axis_index("s")
        r0 = sid * RPT
        pltpu.sync_copy(z_hbm.at[pl.ds(r0, RPT)], acc.at[pl.ds(r0, RPT)])
        tile = cid * 16 + sid
        pltpu.sync_copy(row_hbm.at[pl.ds(tile * NCH, NCH)], ridx)
        pltpu.sync_copy(col_hbm.at[pl.ds(tile * NCH, NCH)], sidx)
        plsc.subcore_barrier()

        def g_start(j, b):
            pltpu.async_copy(y_hbm.at[ridx.at[j]], bufs[b], gsem[b])

        def g_wait(j, b):
            pltpu.make_async_copy(y_hbm.at[ridx.at[j]], bufs[b], gsem[b]).wait()

        def s_start(j, b):
            pltpu.async_copy(bufs[b], acc.at[sidx.at[j]], ssem[b], add=True)

        def s_wait(j, b):
            pltpu.make_async_copy(bufs[b], acc.at[sidx.at[j]], ssem[b]).wait()

        for b in range(NBUF):
            g_start(b, b)

        @pl.loop(0, NGRP)
        def _(k):
            j0 = NBUF * k
            for b in range(NBUF):
                g_wait(j0 + b, b)
                s_start(j0 + b, b)
            for b in range(NBUF):
                jn = j0 + b + NBUF

                @pl.when(jn < NCH)
                def _():
                    s_wait(j0 + b, b)
                    g_start(jn, b)

        for b in range(NBUF):
            s_wait(NCH - NBUF + b, b)

        plsc.subcore_barrier()
        pltpu.sync_copy(acc.at[pl.ds(r0, RPT)], out_hbm.at[cid, pl.ds(r0, RPT)])

    return agg


def _deg_hist(col2, z16):
    mesh = plsc.VectorSubcoreMesh(core_axis_name="c", subcore_axis_name="s")

    @functools.partial(
        pl.kernel,
        out_type=jax.ShapeDtypeStruct((2, NPAD, 16), jnp.float32),
        mesh=mesh,
        scratch_types=[
            pltpu.VMEM((NCH, CHUNK), jnp.int32),
            pltpu.VMEM((CHUNK, 16), jnp.float32),
            pltpu.VMEM_SHARED((NPAD, 16), jnp.float32),
            pltpu.SemaphoreType.DMA,
        ],
        compiler_params=pltpu.CompilerParams(use_tc_tiling_on_sc=False),
    )
    def deg(col_hbm, z_hbm, out_hbm, sidx, ones, acc, sem):
        cid = lax.axis_index("c")
        sid = lax.axis_index("s")
        r0 = sid * RPT
        pltpu.sync_copy(z_hbm.at[pl.ds(r0, RPT)], acc.at[pl.ds(r0, RPT)])
        tile = cid * 16 + sid
        pltpu.sync_copy(col_hbm.at[pl.ds(tile * NCH, NCH)], sidx)

        @pl.loop(0, CHUNK)
        def _(i):
            ones[i, :] = jnp.ones((16,), jnp.float32)

        plsc.subcore_barrier()

        @pl.loop(0, NCH)
        def _(j):
            pltpu.async_copy(ones, acc.at[sidx.at[j]], sem, add=True)

        @pl.loop(0, NCH)
        def _(j):
            pltpu.make_async_copy(ones, acc.at[sidx.at[j]], sem).wait()

        plsc.subcore_barrier()
        pltpu.sync_copy(acc.at[pl.ds(r0, RPT)], out_hbm.at[cid, pl.ds(r0, RPT)])

    return deg(col2, z16)


def _f32_dot(a, b):
    return jnp.dot(a, b, preferred_element_type=jnp.float32)


def _dinv(d_ref):
    d = (d_ref[0] + d_ref[1] + 1.0)[:, 0:1]
    return lax.rsqrt(d)


def _tc_xw1(x, W1, degp):
    def body(x_ref, w_ref, d_ref, o_ref):
        o_ref[...] = _f32_dot(x_ref[...], w_ref[...]) * _dinv(d_ref)

    return pl.pallas_call(
        body,
        grid=(N // BLK,),
        in_specs=[
            pl.BlockSpec((BLK, 128), lambda i: (i, 0)),
            pl.BlockSpec((128, 64), lambda i: (0, 0)),
            pl.BlockSpec((2, BLK, 16), lambda i: (0, i, 0)),
        ],
        out_specs=pl.BlockSpec((BLK, 64), lambda i: (i, 0)),
        out_shape=jax.ShapeDtypeStruct((N, 64), jnp.float32),
    )(x, W1, degp)


def _tc_mid(aggp, y1, degp, b1r, W2p):
    def body(a_ref, y_ref, d_ref, b_ref, w_ref, o_ref):
        dinv = _dinv(d_ref)
        h = jnp.maximum(dinv * (a_ref[0] + a_ref[1] + y_ref[...]) + b_ref[...],
                        0.0)
        o_ref[...] = dinv * _f32_dot(h, w_ref[...])

    return pl.pallas_call(
        body,
        grid=(N // BLK,),
        in_specs=[
            pl.BlockSpec((2, BLK, 64), lambda i: (0, i, 0)),
            pl.BlockSpec((BLK, 64), lambda i: (i, 0)),
            pl.BlockSpec((2, BLK, 16), lambda i: (0, i, 0)),
            pl.BlockSpec((1, 64), lambda i: (0, 0)),
            pl.BlockSpec((64, 16), lambda i: (0, 0)),
        ],
        out_specs=pl.BlockSpec((BLK, 16), lambda i: (i, 0)),
        out_shape=jax.ShapeDtypeStruct((N, 16), jnp.float32),
    )(aggp, y1, degp, b1r, W2p)


def _tc_fin(agg2p, y2p, degp, b2p):
    def body(a_ref, y_ref, d_ref, b_ref, o_ref):
        s = _dinv(d_ref) * (a_ref[0] + a_ref[1] + y_ref[...]) + b_ref[...]
        o_ref[...] = s[:, 0:2]

    return pl.pallas_call(
        body,
        grid=(N // BLK,),
        in_specs=[
            pl.BlockSpec((2, BLK, 16), lambda i: (0, i, 0)),
            pl.BlockSpec((BLK, 16), lambda i: (i, 0)),
            pl.BlockSpec((2, BLK, 16), lambda i: (0, i, 0)),
            pl.BlockSpec((1, 16), lambda i: (0, 0)),
        ],
        out_specs=pl.BlockSpec((BLK, 2), lambda i: (i, 0)),
        out_shape=jax.ShapeDtypeStruct((N, 2), jnp.float32),
    )(agg2p, y2p, degp, b2p)


def kernel(x, edge_index, W1, b1, W2, b2):
    row2 = edge_index[0].astype(jnp.int32).reshape(E // CHUNK, CHUNK)
    col2 = edge_index[1].astype(jnp.int32).reshape(E // CHUNK, CHUNK)

    z16 = jnp.zeros((NPAD, 16), jnp.float32)
    z64 = jnp.zeros((NPAD, 64), jnp.float32)
    W2p = jnp.zeros((64, 16), jnp.float32).at[:, 0:2].set(W2)
    b1r = b1.reshape(1, 64)
    b2p = jnp.zeros((1, 16), jnp.float32).at[0, 0:2].set(b2)

    degp = _deg_hist(col2, z16)
    y1 = _tc_xw1(x, W1, degp)
    agg1 = _make_agg(64)(y1, row2, col2, z64)
    y2p = _tc_mid(agg1, y1, degp, b1r, W2p)
    agg2 = _make_agg(16)(y2p, row2, col2, z16)
    return _tc_fin(agg2, y2p, degp, b2p)


# speedup vs baseline: 49.9908x; 1.0567x over previous
import functools

import jax
import jax.numpy as jnp
from jax import lax
from jax.experimental import pallas as pl
from jax.experimental.pallas import tpu as pltpu
from jax.experimental.pallas import tpu_sc as plsc

N = 10000
E = 320000
NTILES = 32
CHUNK = 80
EPT = E // NTILES
NCH = EPT // CHUNK
NBUF = 5
NGRP = NCH // NBUF
NPAD = 10240
RPT = NPAD // 16
BLK = 1000


def _make_agg(feat):
    mesh = plsc.VectorSubcoreMesh(core_axis_name="c", subcore_axis_name="s")

    @functools.partial(
        pl.kernel,
        out_type=jax.ShapeDtypeStruct((2, NPAD, feat), jnp.float32),
        mesh=mesh,
        scratch_types=[
            pltpu.VMEM((NCH, CHUNK), jnp.int32),
            pltpu.VMEM((NCH, CHUNK), jnp.int32),
        ] + [pltpu.VMEM((CHUNK, feat), jnp.float32)] * NBUF
          + [pltpu.SemaphoreType.DMA] * NBUF
          + [pltpu.SemaphoreType.DMA] * NBUF
          + [pltpu.VMEM_SHARED((NPAD, feat), jnp.float32)],
        compiler_params=pltpu.CompilerParams(use_tc_tiling_on_sc=False),
    )
    def agg(y_hbm, ei_hbm, z_hbm, out_hbm, ridx, sidx, *scr):
        bufs = scr[:NBUF]
        gsem = scr[NBUF:2 * NBUF]
        ssem = scr[2 * NBUF:3 * NBUF]
        acc = scr[3 * NBUF]
        cid = lax.axis_index("c")
        sid = lax.axis_index("s")
        r0 = sid * RPT
        pltpu.sync_copy(z_hbm.at[pl.ds(r0, RPT)], acc.at[pl.ds(r0, RPT)])
        tile = cid * 16 + sid
        pltpu.sync_copy(ei_hbm.at[0, pl.ds(tile * NCH, NCH)], ridx)
        pltpu.sync_copy(ei_hbm.at[1, pl.ds(tile * NCH, NCH)], sidx)
        plsc.subcore_barrier()

        def g_start(j, b):
            pltpu.async_copy(y_hbm.at[ridx.at[j]], bufs[b], gsem[b])

        def g_wait(j, b):
            pltpu.make_async_copy(y_hbm.at[ridx.at[j]], bufs[b], gsem[b]).wait()

        def s_start(j, b):
            pltpu.async_copy(bufs[b], acc.at[sidx.at[j]], ssem[b], add=True)

        def s_wait(j, b):
            pltpu.make_async_copy(bufs[b], acc.at[sidx.at[j]], ssem[b]).wait()

        for b in range(NBUF):
            g_start(b, b)

        @pl.loop(0, NGRP)
        def _(k):
            j0 = NBUF * k
            for b in range(NBUF):
                g_wait(j0 + b, b)
                s_start(j0 + b, b)
            for b in range(NBUF):
                jn = j0 + b + NBUF

                @pl.when(jn < NCH)
                def _():
                    s_wait(j0 + b, b)
                    g_start(jn, b)

        for b in range(NBUF):
            s_wait(NCH - NBUF + b, b)

        plsc.subcore_barrier()
        pltpu.sync_copy(acc.at[pl.ds(r0, RPT)], out_hbm.at[cid, pl.ds(r0, RPT)])

    return agg


def _deg_hist(ei3, z16):
    mesh = plsc.VectorSubcoreMesh(core_axis_name="c", subcore_axis_name="s")

    @functools.partial(
        pl.kernel,
        out_type=jax.ShapeDtypeStruct((2, NPAD, 16), jnp.float32),
        mesh=mesh,
        scratch_types=[
            pltpu.VMEM((NCH, CHUNK), jnp.int32),
            pltpu.VMEM((CHUNK, 16), jnp.float32),
            pltpu.VMEM_SHARED((NPAD, 16), jnp.float32),
            pltpu.SemaphoreType.DMA,
        ],
        compiler_params=pltpu.CompilerParams(use_tc_tiling_on_sc=False),
    )
    def deg(ei_hbm, z_hbm, out_hbm, sidx, ones, acc, sem):
        cid = lax.axis_index("c")
        sid = lax.axis_index("s")
        r0 = sid * RPT
        pltpu.sync_copy(z_hbm.at[pl.ds(r0, RPT)], acc.at[pl.ds(r0, RPT)])
        tile = cid * 16 + sid
        pltpu.sync_copy(ei_hbm.at[1, pl.ds(tile * NCH, NCH)], sidx)

        @pl.loop(0, CHUNK)
        def _(i):
            ones[i, :] = jnp.ones((16,), jnp.float32)

        plsc.subcore_barrier()

        @pl.loop(0, NCH)
        def _(j):
            pltpu.async_copy(ones, acc.at[sidx.at[j]], sem, add=True)

        @pl.loop(0, NCH)
        def _(j):
            pltpu.make_async_copy(ones, acc.at[sidx.at[j]], sem).wait()

        plsc.subcore_barrier()
        pltpu.sync_copy(acc.at[pl.ds(r0, RPT)], out_hbm.at[cid, pl.ds(r0, RPT)])

    return deg(ei3, z16)


def _f32_dot(a, b):
    return jnp.dot(a, b, preferred_element_type=jnp.float32)


def _dinv(d_ref):
    d = (d_ref[0] + d_ref[1] + 1.0)[:, 0:1]
    return lax.rsqrt(d)


def _tc_xw1(x, W1, degp):
    def body(x_ref, w_ref, d_ref, o_ref):
        o_ref[...] = _f32_dot(x_ref[...], w_ref[...]) * _dinv(d_ref)

    return pl.pallas_call(
        body,
        grid=(N // BLK,),
        in_specs=[
            pl.BlockSpec((BLK, 128), lambda i: (i, 0)),
            pl.BlockSpec((128, 64), lambda i: (0, 0)),
            pl.BlockSpec((2, BLK, 16), lambda i: (0, i, 0)),
        ],
        out_specs=pl.BlockSpec((BLK, 64), lambda i: (i, 0)),
        out_shape=jax.ShapeDtypeStruct((N, 64), jnp.float32),
    )(x, W1, degp)


def _tc_mid(aggp, y1, degp, b1r, W2p):
    def body(a_ref, y_ref, d_ref, b_ref, w_ref, o_ref):
        dinv = _dinv(d_ref)
        h = jnp.maximum(dinv * (a_ref[0] + a_ref[1] + y_ref[...]) + b_ref[...],
                        0.0)
        o_ref[...] = dinv * _f32_dot(h, w_ref[...])

    return pl.pallas_call(
        body,
        grid=(N // BLK,),
        in_specs=[
            pl.BlockSpec((2, BLK, 64), lambda i: (0, i, 0)),
            pl.BlockSpec((BLK, 64), lambda i: (i, 0)),
            pl.BlockSpec((2, BLK, 16), lambda i: (0, i, 0)),
            pl.BlockSpec((1, 64), lambda i: (0, 0)),
            pl.BlockSpec((64, 16), lambda i: (0, 0)),
        ],
        out_specs=pl.BlockSpec((BLK, 16), lambda i: (i, 0)),
        out_shape=jax.ShapeDtypeStruct((N, 16), jnp.float32),
    )(aggp, y1, degp, b1r, W2p)


def _tc_fin(agg2p, y2p, degp, b2p):
    def body(a_ref, y_ref, d_ref, b_ref, o_ref):
        s = _dinv(d_ref) * (a_ref[0] + a_ref[1] + y_ref[...]) + b_ref[...]
        o_ref[...] = s[:, 0:2]

    return pl.pallas_call(
        body,
        grid=(N // BLK,),
        in_specs=[
            pl.BlockSpec((2, BLK, 16), lambda i: (0, i, 0)),
            pl.BlockSpec((BLK, 16), lambda i: (i, 0)),
            pl.BlockSpec((2, BLK, 16), lambda i: (0, i, 0)),
            pl.BlockSpec((1, 16), lambda i: (0, 0)),
        ],
        out_specs=pl.BlockSpec((BLK, 2), lambda i: (i, 0)),
        out_shape=jax.ShapeDtypeStruct((N, 2), jnp.float32),
    )(agg2p, y2p, degp, b2p)


def kernel(x, edge_index, W1, b1, W2, b2):
    ei3 = edge_index.astype(jnp.int32).reshape(2, E // CHUNK, CHUNK)

    z16 = jnp.zeros((NPAD, 16), jnp.float32)
    z64 = jnp.zeros((NPAD, 64), jnp.float32)
    W2p = jnp.zeros((64, 16), jnp.float32).at[:, 0:2].set(W2)
    b1r = b1.reshape(1, 64)
    b2p = jnp.zeros((1, 16), jnp.float32).at[0, 0:2].set(b2)

    degp = _deg_hist(ei3, z16)
    y1 = _tc_xw1(x, W1, degp)
    agg1 = _make_agg(64)(y1, ei3, z64)
    y2p = _tc_mid(agg1, y1, degp, b1r, W2p)
    agg2 = _make_agg(16)(y2p, ei3, z16)
    return _tc_fin(agg2, y2p, degp, b2p)


# speedup vs baseline: 54.6979x; 1.0942x over previous
import functools

import jax
import jax.numpy as jnp
from jax import lax
from jax.experimental import pallas as pl
from jax.experimental.pallas import tpu as pltpu
from jax.experimental.pallas import tpu_sc as plsc

N = 10000
E = 320000
NTILES = 32
CHUNK = 80
EPT = E // NTILES
NCH = EPT // CHUNK
NBUF = 5
NGRP = NCH // NBUF
NPAD = 10240
RPT = NPAD // 16
BLK = 1000


def _make_agg(feat):
    mesh = plsc.VectorSubcoreMesh(core_axis_name="c", subcore_axis_name="s")

    @functools.partial(
        pl.kernel,
        out_type=jax.ShapeDtypeStruct((2, NPAD, feat), jnp.float32),
        mesh=mesh,
        scratch_types=[
            pltpu.VMEM((NCH, CHUNK), jnp.int32),
            pltpu.VMEM((NCH, CHUNK), jnp.int32),
        ] + [pltpu.VMEM((CHUNK, feat), jnp.float32)] * (2 * NBUF)
          + [pltpu.SemaphoreType.DMA] * (2 * NBUF)
          + [pltpu.SemaphoreType.DMA] * (2 * NBUF)
          + [pltpu.VMEM_SHARED((NPAD, feat), jnp.float32)],
        compiler_params=pltpu.CompilerParams(use_tc_tiling_on_sc=False),
    )
    def agg(y_hbm, ei_hbm, z_hbm, out_hbm, ridx, sidx, *scr):
        nb = 2 * NBUF
        bufs = scr[:nb]
        gsem = scr[nb:2 * nb]
        ssem = scr[2 * nb:3 * nb]
        acc = scr[3 * nb]
        cid = lax.axis_index("c")
        sid = lax.axis_index("s")
        r0 = sid * RPT
        pltpu.sync_copy(z_hbm.at[pl.ds(r0, RPT)], acc.at[pl.ds(r0, RPT)])
        tile = cid * 16 + sid
        pltpu.sync_copy(ei_hbm.at[0, pl.ds(tile * NCH, NCH)], ridx)
        pltpu.sync_copy(ei_hbm.at[1, pl.ds(tile * NCH, NCH)], sidx)
        plsc.subcore_barrier()

        def g_start(j, b):
            pltpu.async_copy(y_hbm.at[ridx.at[j]], bufs[b], gsem[b])

        def g_wait(j, b):
            pltpu.make_async_copy(y_hbm.at[ridx.at[j]], bufs[b], gsem[b]).wait()

        def s_start(j, b):
            pltpu.async_copy(bufs[b], acc.at[sidx.at[j]], ssem[b], add=True)

        def s_wait(j, b):
            pltpu.make_async_copy(bufs[b], acc.at[sidx.at[j]], ssem[b]).wait()

        for b in range(2 * NBUF):
            g_start(b, b)

        @pl.loop(0, (NCH - NBUF) // (2 * NBUF))
        def _(kk):
            j0 = 2 * NBUF * kk
            for h in range(2):
                for i in range(NBUF):
                    j = j0 + h * NBUF + i
                    b = h * NBUF + i
                    g_wait(j, b)
                    s_start(j, b)
                for i in range(NBUF):
                    j = j0 + h * NBUF + i
                    b = h * NBUF + i
                    jn = j + 2 * NBUF

                    @pl.when(jn < NCH)
                    def _():
                        s_wait(j, b)
                        g_start(jn, b)

        for i in range(NBUF):
            j = NCH - NBUF + i
            g_wait(j, i)
            s_start(j, i)
        for i in range(NBUF):
            s_wait(NCH - 2 * NBUF + i, NBUF + i)
        for i in range(NBUF):
            s_wait(NCH - NBUF + i, i)

        plsc.subcore_barrier()
        pltpu.sync_copy(acc.at[pl.ds(r0, RPT)], out_hbm.at[cid, pl.ds(r0, RPT)])

    return agg


def _deg_hist(ei3, z16):
    mesh = plsc.VectorSubcoreMesh(core_axis_name="c", subcore_axis_name="s")

    @functools.partial(
        pl.kernel,
        out_type=jax.ShapeDtypeStruct((2, NPAD, 16), jnp.float32),
        mesh=mesh,
        scratch_types=[
            pltpu.VMEM((NCH, CHUNK), jnp.int32),
            pltpu.VMEM((CHUNK, 16), jnp.float32),
            pltpu.VMEM_SHARED((NPAD, 16), jnp.float32),
            pltpu.SemaphoreType.DMA,
        ],
        compiler_params=pltpu.CompilerParams(use_tc_tiling_on_sc=False),
    )
    def deg(ei_hbm, z_hbm, out_hbm, sidx, ones, acc, sem):
        cid = lax.axis_index("c")
        sid = lax.axis_index("s")
        r0 = sid * RPT
        pltpu.sync_copy(z_hbm.at[pl.ds(r0, RPT)], acc.at[pl.ds(r0, RPT)])
        tile = cid * 16 + sid
        pltpu.sync_copy(ei_hbm.at[1, pl.ds(tile * NCH, NCH)], sidx)

        @pl.loop(0, CHUNK)
        def _(i):
            ones[i, :] = jnp.ones((16,), jnp.float32)

        plsc.subcore_barrier()

        @pl.loop(0, NCH)
        def _(j):
            pltpu.async_copy(ones, acc.at[sidx.at[j]], sem, add=True)

        @pl.loop(0, NCH)
        def _(j):
            pltpu.make_async_copy(ones, acc.at[sidx.at[j]], sem).wait()

        plsc.subcore_barrier()
        pltpu.sync_copy(acc.at[pl.ds(r0, RPT)], out_hbm.at[cid, pl.ds(r0, RPT)])

    return deg(ei3, z16)


def _f32_dot(a, b):
    return jnp.dot(a, b, preferred_element_type=jnp.float32)


def _dinv(d_ref):
    d = (d_ref[0] + d_ref[1] + 1.0)[:, 0:1]
    return lax.rsqrt(d)


def _tc_xw1(x, W1, degp):
    def body(x_ref, w_ref, d_ref, o_ref):
        o_ref[...] = _f32_dot(x_ref[...], w_ref[...]) * _dinv(d_ref)

    return pl.pallas_call(
        body,
        grid=(N // BLK,),
        in_specs=[
            pl.BlockSpec((BLK, 128), lambda i: (i, 0)),
            pl.BlockSpec((128, 64), lambda i: (0, 0)),
            pl.BlockSpec((2, BLK, 16), lambda i: (0, i, 0)),
        ],
        out_specs=pl.BlockSpec((BLK, 64), lambda i: (i, 0)),
        out_shape=jax.ShapeDtypeStruct((N, 64), jnp.float32),
    )(x, W1, degp)


def _tc_mid(aggp, y1, degp, b1r, W2p):
    def body(a_ref, y_ref, d_ref, b_ref, w_ref, o_ref):
        dinv = _dinv(d_ref)
        h = jnp.maximum(dinv * (a_ref[0] + a_ref[1] + y_ref[...]) + b_ref[...],
                        0.0)
        o_ref[...] = dinv * _f32_dot(h, w_ref[...])

    return pl.pallas_call(
        body,
        grid=(N // BLK,),
        in_specs=[
            pl.BlockSpec((2, BLK, 64), lambda i: (0, i, 0)),
            pl.BlockSpec((BLK, 64), lambda i: (i, 0)),
            pl.BlockSpec((2, BLK, 16), lambda i: (0, i, 0)),
            pl.BlockSpec((1, 64), lambda i: (0, 0)),
            pl.BlockSpec((64, 16), lambda i: (0, 0)),
        ],
        out_specs=pl.BlockSpec((BLK, 16), lambda i: (i, 0)),
        out_shape=jax.ShapeDtypeStruct((N, 16), jnp.float32),
    )(aggp, y1, degp, b1r, W2p)


def _tc_fin(agg2p, y2p, degp, b2p):
    def body(a_ref, y_ref, d_ref, b_ref, o_ref):
        s = _dinv(d_ref) * (a_ref[0] + a_ref[1] + y_ref[...]) + b_ref[...]
        o_ref[...] = s[:, 0:2]

    return pl.pallas_call(
        body,
        grid=(N // BLK,),
        in_specs=[
            pl.BlockSpec((2, BLK, 16), lambda i: (0, i, 0)),
            pl.BlockSpec((BLK, 16), lambda i: (i, 0)),
            pl.BlockSpec((2, BLK, 16), lambda i: (0, i, 0)),
            pl.BlockSpec((1, 16), lambda i: (0, 0)),
        ],
        out_specs=pl.BlockSpec((BLK, 2), lambda i: (i, 0)),
        out_shape=jax.ShapeDtypeStruct((N, 2), jnp.float32),
    )(agg2p, y2p, degp, b2p)


def kernel(x, edge_index, W1, b1, W2, b2):
    ei3 = edge_index.astype(jnp.int32).reshape(2, E // CHUNK, CHUNK)

    z16 = jnp.zeros((NPAD, 16), jnp.float32)
    z64 = jnp.zeros((NPAD, 64), jnp.float32)
    W2p = jnp.zeros((64, 16), jnp.float32).at[:, 0:2].set(W2)
    b1r = b1.reshape(1, 64)
    b2p = jnp.zeros((1, 16), jnp.float32).at[0, 0:2].set(b2)

    degp = _deg_hist(ei3, z16)
    y1 = _tc_xw1(x, W1, degp)
    agg1 = _make_agg(64)(y1, ei3, z64)
    y2p = _tc_mid(agg1, y1, degp, b1r, W2p)
    agg2 = _make_agg(16)(y2p, ei3, z16)
    return _tc_fin(agg2, y2p, degp, b2p)


# speedup vs baseline: 54.8206x; 1.0022x over previous
import functools

import jax
import jax.numpy as jnp
from jax import lax
from jax.experimental import pallas as pl
from jax.experimental.pallas import tpu as pltpu
from jax.experimental.pallas import tpu_sc as plsc

N = 10000
E = 320000
NTILES = 32
CHUNK = 80
EPT = E // NTILES
NCH = EPT // CHUNK
NBUF = 5
NPAD = 10240
RPT = NPAD // 16
BLK = 1000


def _make_agg(feat):
    mesh = plsc.VectorSubcoreMesh(core_axis_name="c", subcore_axis_name="s")

    @functools.partial(
        pl.kernel,
        out_type=jax.ShapeDtypeStruct((2, NPAD, feat), jnp.float32),
        mesh=mesh,
        scratch_types=[
            pltpu.VMEM((NCH, CHUNK), jnp.int32),
            pltpu.VMEM((NCH, CHUNK), jnp.int32),
        ] + [pltpu.VMEM((CHUNK, feat), jnp.float32)] * (2 * NBUF)
          + [pltpu.SemaphoreType.DMA] * (2 * NBUF)
          + [pltpu.SemaphoreType.DMA] * (2 * NBUF)
          + [pltpu.VMEM_SHARED((NPAD, feat), jnp.float32)],
        compiler_params=pltpu.CompilerParams(use_tc_tiling_on_sc=False),
    )
    def agg(y_hbm, ei_hbm, z_hbm, out_hbm, ridx, sidx, *scr):
        nb = 2 * NBUF
        bufs = scr[:nb]
        gsem = scr[nb:2 * nb]
        ssem = scr[2 * nb:3 * nb]
        acc = scr[3 * nb]
        cid = lax.axis_index("c")
        sid = lax.axis_index("s")
        r0 = sid * RPT
        pltpu.sync_copy(z_hbm.at[pl.ds(r0, RPT)], acc.at[pl.ds(r0, RPT)])
        tile = cid * 16 + sid
        pltpu.sync_copy(ei_hbm.at[0, pl.ds(tile * NCH, NCH)], ridx)
        pltpu.sync_copy(ei_hbm.at[1, pl.ds(tile * NCH, NCH)], sidx)
        plsc.subcore_barrier()

        def g_start(j, b):
            pltpu.async_copy(y_hbm.at[ridx.at[j]], bufs[b], gsem[b])

        def g_wait(j, b):
            pltpu.make_async_copy(y_hbm.at[ridx.at[j]], bufs[b], gsem[b]).wait()

        def s_start(j, b):
            pltpu.async_copy(bufs[b], acc.at[sidx.at[j]], ssem[b], add=True)

        def s_wait(j, b):
            pltpu.make_async_copy(bufs[b], acc.at[sidx.at[j]], ssem[b]).wait()

        for b in range(2 * NBUF):
            g_start(b, b)

        @pl.loop(0, (NCH - NBUF) // (2 * NBUF))
        def _(kk):
            j0 = 2 * NBUF * kk
            for h in range(2):
                for i in range(NBUF):
                    j = j0 + h * NBUF + i
                    b = h * NBUF + i
                    g_wait(j, b)
                    s_start(j, b)
                for i in range(NBUF):
                    j = j0 + h * NBUF + i
                    b = h * NBUF + i
                    jn = j + 2 * NBUF

                    @pl.when(jn < NCH)
                    def _():
                        s_wait(j, b)
                        g_start(jn, b)

        for i in range(NBUF):
            j = NCH - NBUF + i
            g_wait(j, i)
            s_start(j, i)
        for i in range(NBUF):
            s_wait(NCH - 2 * NBUF + i, NBUF + i)
        for i in range(NBUF):
            s_wait(NCH - NBUF + i, i)

        plsc.subcore_barrier()
        pltpu.sync_copy(acc.at[pl.ds(r0, RPT)], out_hbm.at[cid, pl.ds(r0, RPT)])

    return agg


def _deg_hist(ei3, z16):
    mesh = plsc.VectorSubcoreMesh(core_axis_name="c", subcore_axis_name="s")

    @functools.partial(
        pl.kernel,
        out_type=jax.ShapeDtypeStruct((2, NPAD, 16), jnp.float32),
        mesh=mesh,
        scratch_types=[
            pltpu.VMEM((NCH, CHUNK), jnp.int32),
            pltpu.VMEM((CHUNK, 16), jnp.float32),
            pltpu.VMEM_SHARED((NPAD, 16), jnp.float32),
            pltpu.SemaphoreType.DMA,
        ],
        compiler_params=pltpu.CompilerParams(use_tc_tiling_on_sc=False),
    )
    def deg(ei_hbm, z_hbm, out_hbm, sidx, ones, acc, sem):
        cid = lax.axis_index("c")
        sid = lax.axis_index("s")
        r0 = sid * RPT
        pltpu.sync_copy(z_hbm.at[pl.ds(r0, RPT)], acc.at[pl.ds(r0, RPT)])
        tile = cid * 16 + sid
        pltpu.sync_copy(ei_hbm.at[1, pl.ds(tile * NCH, NCH)], sidx)

        @pl.loop(0, CHUNK)
        def _(i):
            ones[i, :] = jnp.ones((16,), jnp.float32)

        plsc.subcore_barrier()

        @pl.loop(0, NCH)
        def _(j):
            pltpu.async_copy(ones, acc.at[sidx.at[j]], sem, add=True)

        @pl.loop(0, NCH)
        def _(j):
            pltpu.make_async_copy(ones, acc.at[sidx.at[j]], sem).wait()

        plsc.subcore_barrier()
        pltpu.sync_copy(acc.at[pl.ds(r0, RPT)], out_hbm.at[cid, pl.ds(r0, RPT)])

    return deg(ei3, z16)


def _f32_dot(a, b):
    return jnp.dot(a, b, preferred_element_type=jnp.float32)


def _dinv(d_ref):
    d = (d_ref[0] + d_ref[1] + 1.0)[:, 0:1]
    return lax.rsqrt(d)


def _tc_xw1(x, W1, degp):
    def body(x_ref, w_ref, d_ref, o_ref):
        o_ref[...] = _f32_dot(x_ref[...], w_ref[...]) * _dinv(d_ref)

    return pl.pallas_call(
        body,
        grid=(N // BLK,),
        in_specs=[
            pl.BlockSpec((BLK, 128), lambda i: (i, 0)),
            pl.BlockSpec((128, 64), lambda i: (0, 0)),
            pl.BlockSpec((2, BLK, 16), lambda i: (0, i, 0)),
        ],
        out_specs=pl.BlockSpec((BLK, 64), lambda i: (i, 0)),
        out_shape=jax.ShapeDtypeStruct((N, 64), jnp.float32),
    )(x, W1, degp)


def _tc_mid(aggp, y1, degp, b1r, W2p):
    def body(a_ref, y_ref, d_ref, b_ref, w_ref, o_ref):
        dinv = _dinv(d_ref)
        h = jnp.maximum(dinv * (a_ref[0] + a_ref[1] + y_ref[...]) + b_ref[...],
                        0.0)
        o_ref[...] = dinv * _f32_dot(h, w_ref[...])

    return pl.pallas_call(
        body,
        grid=(N // BLK,),
        in_specs=[
            pl.BlockSpec((2, BLK, 64), lambda i: (0, i, 0)),
            pl.BlockSpec((BLK, 64), lambda i: (i, 0)),
            pl.BlockSpec((2, BLK, 16), lambda i: (0, i, 0)),
            pl.BlockSpec((1, 64), lambda i: (0, 0)),
            pl.BlockSpec((64, 16), lambda i: (0, 0)),
        ],
        out_specs=pl.BlockSpec((BLK, 16), lambda i: (i, 0)),
        out_shape=jax.ShapeDtypeStruct((N, 16), jnp.float32),
    )(aggp, y1, degp, b1r, W2p)


def _tc_fin(agg2p, y2p, degp, b2p):
    def body(a_ref, y_ref, d_ref, b_ref, o_ref):
        s = _dinv(d_ref) * (a_ref[0] + a_ref[1] + y_ref[...]) + b_ref[...]
        o_ref[...] = s[:, 0:2]

    return pl.pallas_call(
        body,
        grid=(N // BLK,),
        in_specs=[
            pl.BlockSpec((2, BLK, 16), lambda i: (0, i, 0)),
            pl.BlockSpec((BLK, 16), lambda i: (i, 0)),
            pl.BlockSpec((2, BLK, 16), lambda i: (0, i, 0)),
            pl.BlockSpec((1, 16), lambda i: (0, 0)),
        ],
        out_specs=pl.BlockSpec((BLK, 2), lambda i: (i, 0)),
        out_shape=jax.ShapeDtypeStruct((N, 2), jnp.float32),
    )(agg2p, y2p, degp, b2p)


def kernel(x, edge_index, W1, b1, W2, b2):
    ei3 = edge_index.astype(jnp.int32).reshape(2, E // CHUNK, CHUNK)

    z16 = jnp.zeros((NPAD, 16), jnp.float32)
    z64 = jnp.zeros((NPAD, 64), jnp.float32)
    W2p = jnp.zeros((64, 16), jnp.float32).at[:, 0:2].set(W2)
    b1r = b1.reshape(1, 64)
    b2p = jnp.zeros((1, 16), jnp.float32).at[0, 0:2].set(b2)

    degp = _deg_hist(ei3, z16)
    y1 = _tc_xw1(x, W1, degp)
    agg1 = _make_agg(64)(y1, ei3, z64)
    y2p = _tc_mid(agg1, y1, degp, b1r, W2p)
    agg2 = _make_agg(16)(y2p, ei3, z16)
    return _tc_fin(agg2, y2p, degp, b2p)
